# Initial kernel scaffold; baseline (speedup 1.0000x reference)
import jax
import jax.numpy as jnp
from jax.experimental import pallas as pl


def kernel(x, node_types, type_table):
    raise NotImplementedError("write your pallas kernel here")


# speedup vs baseline: 1.1692x; 1.1692x over previous
import functools

import jax
import jax.numpy as jnp
from jax import lax
from jax.experimental import pallas as pl
from jax.experimental.pallas import tpu as pltpu
from jax.experimental.pallas import tpu_sc as plsc

N_NODES = 100000
D = 128
NC = 2
NS = 16
NW = NC * NS

ROWS_W = 3120
MAIN = NW * ROWS_W
CH = 120
NCHUNK = ROWS_W // CH
TAIL = N_NODES - MAIN
TAIL_W = TAIL // 8


def _add_rows(xb, gb, nrows):
    def row(r, carry):
        for j in range(D // 16):
            plsc.addupdate(xb.at[r, pl.ds(j * 16, 16)], gb[r, pl.ds(j * 16, 16)])
        return carry

    lax.fori_loop(0, nrows, row, None)


def _body(x_hbm, idx_hbm, tbl_hbm, out_hbm, idx_v, xb, gb, tidx_v, sem):
    wid = lax.axis_index("s") * NC + lax.axis_index("c")
    base = wid * ROWS_W

    pltpu.sync_copy(idx_hbm.at[pl.ds(base, ROWS_W)], idx_v)

    def chunk(k, carry):
        r0 = base + k * CH
        pltpu.sync_copy(x_hbm.at[pl.ds(r0, CH)], xb)
        pltpu.async_copy(tbl_hbm.at[idx_v.at[pl.ds(k * CH, CH)]], gb, sem).wait()
        _add_rows(xb, gb, CH)
        pltpu.sync_copy(xb, out_hbm.at[pl.ds(r0, CH)])
        return carry

    lax.fori_loop(0, NCHUNK, chunk, None)

    @pl.when(wid < TAIL_W)
    def _tail():
        t0 = MAIN + wid * 8
        pltpu.sync_copy(idx_hbm.at[pl.ds(t0, 8)], tidx_v)
        pltpu.sync_copy(x_hbm.at[pl.ds(t0, 8)], xb.at[pl.ds(0, 8)])
        pltpu.async_copy(tbl_hbm.at[tidx_v], gb.at[pl.ds(0, 8)], sem).wait()
        _add_rows(xb, gb, 8)
        pltpu.sync_copy(xb.at[pl.ds(0, 8)], out_hbm.at[pl.ds(t0, 8)])


@jax.jit
def _sc_add_embed(x, idx, tbl):
    mesh = plsc.VectorSubcoreMesh(
        core_axis_name="c", subcore_axis_name="s", num_cores=NC, num_subcores=NS
    )
    return pl.kernel(
        _body,
        out_type=jax.ShapeDtypeStruct((N_NODES, D), jnp.float32),
        mesh=mesh,
        scratch_types=[
            pltpu.VMEM((ROWS_W,), jnp.int32),
            pltpu.VMEM((CH, D), jnp.float32),
            pltpu.VMEM((CH, D), jnp.float32),
            pltpu.VMEM((8,), jnp.int32),
            pltpu.SemaphoreType.DMA,
        ],
    )(x, idx, tbl)


def kernel(x, node_types, type_table):
    return _sc_add_embed(x, node_types.astype(jnp.int32), type_table)


# speedup vs baseline: 1.2193x; 1.0428x over previous
import jax
import jax.numpy as jnp
from jax import lax
from jax.experimental import pallas as pl
from jax.experimental.pallas import tpu as pltpu
from jax.experimental.pallas import tpu_sc as plsc

N_NODES = 100000
D = 128
NC = 2
NS = 16
NW = NC * NS

ROWS_W = 3120
MAIN = NW * ROWS_W
CH = 120
NCHUNK = ROWS_W // CH
NBUF = 3
TAIL = N_NODES - MAIN
TAIL_W = TAIL // 8


def _add_rows(xb, gb, nrows):
    def row(r, carry):
        for j in range(D // 16):
            plsc.addupdate(xb.at[r, pl.ds(j * 16, 16)], gb[r, pl.ds(j * 16, 16)])
        return carry

    lax.fori_loop(0, nrows, row, None)


def _body(x_hbm, idx_hbm, tbl_hbm, out_hbm, idx_v,
          xb0, xb1, xb2, gb0, gb1, gb2, txb, tgb, tidx_v,
          sx0, sx1, sx2, sg0, sg1, sg2, so0, so1, so2, st):
    xb = [xb0, xb1, xb2]
    gb = [gb0, gb1, gb2]
    sx = [sx0, sx1, sx2]
    sg = [sg0, sg1, sg2]
    so = [so0, so1, so2]

    wid = lax.axis_index("s") * NC + lax.axis_index("c")
    base = wid * ROWS_W

    pltpu.sync_copy(idx_hbm.at[pl.ds(base, ROWS_W)], idx_v)

    def start_in(k):
        s = k % NBUF
        r0 = base + k * CH
        dx = pltpu.async_copy(x_hbm.at[pl.ds(r0, CH)], xb[s], sx[s])
        dg = pltpu.async_copy(tbl_hbm.at[idx_v.at[pl.ds(k * CH, CH)]], gb[s], sg[s])
        return dx, dg

    in_desc = [None] * NCHUNK
    out_desc = [None] * NCHUNK
    in_desc[0] = start_in(0)
    in_desc[1] = start_in(1)
    for k in range(NCHUNK):
        s = k % NBUF
        dx, dg = in_desc[k]
        dx.wait()
        dg.wait()
        _add_rows(xb[s], gb[s], CH)
        out_desc[k] = pltpu.async_copy(
            xb[s], out_hbm.at[pl.ds(base + k * CH, CH)], so[s])
        if k + 2 < NCHUNK:
            if k - 1 >= 0:
                out_desc[k - 1].wait()
            in_desc[k + 2] = start_in(k + 2)
    for k in range(max(0, NCHUNK - 3), NCHUNK):
        out_desc[k].wait()

    @pl.when(wid < TAIL_W)
    def _tail():
        t0 = MAIN + wid * 8
        pltpu.sync_copy(idx_hbm.at[pl.ds(t0, 8)], tidx_v)
        pltpu.sync_copy(x_hbm.at[pl.ds(t0, 8)], txb)
        pltpu.async_copy(tbl_hbm.at[tidx_v], tgb, st).wait()
        _add_rows(txb, tgb, 8)
        pltpu.sync_copy(txb, out_hbm.at[pl.ds(t0, 8)])


@jax.jit
def _sc_add_embed(x, idx, tbl):
    mesh = plsc.VectorSubcoreMesh(
        core_axis_name="c", subcore_axis_name="s", num_cores=NC, num_subcores=NS
    )
    return pl.kernel(
        _body,
        out_type=jax.ShapeDtypeStruct((N_NODES, D), jnp.float32),
        mesh=mesh,
        scratch_types=[
            pltpu.VMEM((ROWS_W,), jnp.int32),
            pltpu.VMEM((CH, D), jnp.float32),
            pltpu.VMEM((CH, D), jnp.float32),
            pltpu.VMEM((CH, D), jnp.float32),
            pltpu.VMEM((CH, D), jnp.float32),
            pltpu.VMEM((CH, D), jnp.float32),
            pltpu.VMEM((CH, D), jnp.float32),
            pltpu.VMEM((8, D), jnp.float32),
            pltpu.VMEM((8, D), jnp.float32),
            pltpu.VMEM((8,), jnp.int32),
            pltpu.SemaphoreType.DMA,
            pltpu.SemaphoreType.DMA,
            pltpu.SemaphoreType.DMA,
            pltpu.SemaphoreType.DMA,
            pltpu.SemaphoreType.DMA,
            pltpu.SemaphoreType.DMA,
            pltpu.SemaphoreType.DMA,
            pltpu.SemaphoreType.DMA,
            pltpu.SemaphoreType.DMA,
            pltpu.SemaphoreType.DMA,
        ],
    )(x, idx, tbl)


def kernel(x, node_types, type_table):
    return _sc_add_embed(x, node_types.astype(jnp.int32), type_table)


# speedup vs baseline: 1.8771x; 1.5395x over previous
import jax
import jax.numpy as jnp
from jax import lax
from jax.experimental import pallas as pl
from jax.experimental.pallas import tpu as pltpu
from jax.experimental.pallas import tpu_sc as plsc

N_NODES = 100000
D = 128
NC = 2
NS = 16
NW = NC * NS
L = 16

ROWS_W = 3120
MAIN = NW * ROWS_W
CH = 240
NCHUNK = ROWS_W // CH
NBUF = 3
TAIL = N_NODES - MAIN
TAIL_W = TAIL // 8


def _splat_lane(tv, rl):
    dnums = lax.GatherDimensionNumbers(
        offset_dims=(), collapsed_slice_dims=(0,), start_index_map=(0,))
    return lax.gather(
        tv, jnp.full((L, 1), rl, jnp.int32), dnums, (1,),
        mode=lax.GatherScatterMode.PROMISE_IN_BOUNDS)


def _add_group(xb, tbl_v, tv, row0, rows):
    for rl in range(rows):
        t = _splat_lane(tv, rl) * D
        for j in range(D // L):
            col = jnp.arange(j * L, (j + 1) * L, dtype=jnp.int32)
            v = plsc.load_gather(tbl_v, [t + col])
            plsc.addupdate(xb.at[row0 + rl, pl.ds(j * L, L)], v)


def _body(x_hbm, idx_hbm, tbl_hbm, out_hbm, idx_v, tbl_v,
          xb0, xb1, xb2, tidx_v, txb,
          sx0, sx1, sx2, so0, so1, so2, st):
    xb = [xb0, xb1, xb2]
    sx = [sx0, sx1, sx2]
    so = [so0, so1, so2]

    wid = lax.axis_index("s") * NC + lax.axis_index("c")
    base = wid * ROWS_W

    pltpu.sync_copy(tbl_hbm, tbl_v)
    pltpu.sync_copy(idx_hbm.at[pl.ds(base, ROWS_W)], idx_v)

    def start_in(k):
        s = k % NBUF
        return pltpu.async_copy(
            x_hbm.at[pl.ds(base + k * CH, CH)], xb[s], sx[s])

    in_desc = [None] * NCHUNK
    out_desc = [None] * NCHUNK
    in_desc[0] = start_in(0)
    in_desc[1] = start_in(1)
    for k in range(NCHUNK):
        s = k % NBUF
        in_desc[k].wait()

        def chunk_groups(g, carry, k=k, s=s):
            tv = idx_v[pl.ds(k * CH + g * L, L)]
            _add_group(xb[s], tbl_v, tv, g * L, L)
            return carry

        lax.fori_loop(0, CH // L, chunk_groups, None)
        out_desc[k] = pltpu.async_copy(
            xb[s], out_hbm.at[pl.ds(base + k * CH, CH)], so[s])
        if k + 2 < NCHUNK:
            if k - 1 >= 0:
                out_desc[k - 1].wait()
            in_desc[k + 2] = start_in(k + 2)
    for k in range(max(0, NCHUNK - 3), NCHUNK):
        out_desc[k].wait()

    @pl.when(wid < TAIL_W)
    def _tail():
        t0 = MAIN + wid * 8
        pltpu.sync_copy(idx_hbm.at[pl.ds(t0 - 8, L)], tidx_v)
        pltpu.sync_copy(x_hbm.at[pl.ds(t0, 8)], txb)
        tv = tidx_v[...]
        for rl in range(8):
            t = _splat_lane(tv, 8 + rl) * D
            for j in range(D // L):
                col = jnp.arange(j * L, (j + 1) * L, dtype=jnp.int32)
                v = plsc.load_gather(tbl_v, [t + col])
                plsc.addupdate(txb.at[rl, pl.ds(j * L, L)], v)
        pltpu.sync_copy(txb, out_hbm.at[pl.ds(t0, 8)])


@jax.jit
def _sc_add_embed(x, idx, tbl):
    mesh = plsc.VectorSubcoreMesh(
        core_axis_name="c", subcore_axis_name="s", num_cores=NC, num_subcores=NS
    )
    return pl.kernel(
        _body,
        out_type=jax.ShapeDtypeStruct((N_NODES, D), jnp.float32),
        mesh=mesh,
        compiler_params=pltpu.CompilerParams(needs_layout_passes=False),
        scratch_types=[
            pltpu.VMEM((ROWS_W,), jnp.int32),
            pltpu.VMEM((64 * D,), jnp.float32),
            pltpu.VMEM((CH, D), jnp.float32),
            pltpu.VMEM((CH, D), jnp.float32),
            pltpu.VMEM((CH, D), jnp.float32),
            pltpu.VMEM((L,), jnp.int32),
            pltpu.VMEM((8, D), jnp.float32),
            pltpu.SemaphoreType.DMA,
            pltpu.SemaphoreType.DMA,
            pltpu.SemaphoreType.DMA,
            pltpu.SemaphoreType.DMA,
            pltpu.SemaphoreType.DMA,
            pltpu.SemaphoreType.DMA,
            pltpu.SemaphoreType.DMA,
        ],
    )(x, idx, tbl)


def kernel(x, node_types, type_table):
    return _sc_add_embed(x, node_types.astype(jnp.int32), type_table.reshape(-1))


# speedup vs baseline: 3.4834x; 1.8557x over previous
import jax
import jax.numpy as jnp
from jax import lax
from jax.experimental import pallas as pl
from jax.experimental.pallas import tpu as pltpu
from jax.experimental.pallas import tpu_sc as plsc

N_NODES = 100000
D = 128
NC = 2
NS = 16
NW = NC * NS
L = 16

ROWS_W = 3120
MAIN = NW * ROWS_W
CH = 240
NCHUNK = ROWS_W // CH
NBUF = 3
TAIL = N_NODES - MAIN
TAIL_W = TAIL // 8


def _splat_lane(tv, rl):
    dnums = lax.GatherDimensionNumbers(
        offset_dims=(), collapsed_slice_dims=(0,), start_index_map=(0,))
    return lax.gather(
        tv, jnp.full((L, 1), rl, jnp.int32), dnums, (1,),
        mode=lax.GatherScatterMode.PROMISE_IN_BOUNDS)


def _add_row(xb, tbl_v, t, rowflat):
    for j in range(D // L):
        col = jnp.arange(j * L, (j + 1) * L, dtype=jnp.int32)
        v = plsc.load_gather(tbl_v, [t + col])
        plsc.addupdate(xb.at[pl.ds(rowflat + j * L, L)], v)


def _body(x_hbm, idx_hbm, tbl_hbm, out_hbm, idx_v, tbl_v,
          xb0, xb1, xb2, tidx_v, txb,
          sx0, sx1, sx2, so0, so1, so2, st):
    xb = [xb0, xb1, xb2]
    sx = [sx0, sx1, sx2]
    so = [so0, so1, so2]

    wid = lax.axis_index("s") * NC + lax.axis_index("c")
    base = wid * ROWS_W

    pltpu.sync_copy(tbl_hbm, tbl_v)
    pltpu.sync_copy(idx_hbm.at[pl.ds(base, ROWS_W)], idx_v.at[pl.ds(0, ROWS_W)])

    def start_in(k):
        s = k % NBUF
        return pltpu.async_copy(
            x_hbm.at[pl.ds((base + k * CH) * D, CH * D)], xb[s], sx[s])

    in_desc = [None] * NCHUNK
    out_desc = [None] * NCHUNK
    in_desc[0] = start_in(0)
    in_desc[1] = start_in(1)
    for k in range(NCHUNK):
        s = k % NBUF
        in_desc[k].wait()

        @plsc.parallel_loop(0, CH, unroll=2)
        def chunk_rows(r, k=k, s=s):
            tv = idx_v[pl.ds(k * CH + r, L)]
            t = _splat_lane(tv, 0) * D
            _add_row(xb[s], tbl_v, t, r * D)

        out_desc[k] = pltpu.async_copy(
            xb[s], out_hbm.at[pl.ds((base + k * CH) * D, CH * D)], so[s])
        if k + 2 < NCHUNK:
            if k - 1 >= 0:
                out_desc[k - 1].wait()
            in_desc[k + 2] = start_in(k + 2)
    for k in range(max(0, NCHUNK - 3), NCHUNK):
        out_desc[k].wait()

    @pl.when(wid < TAIL_W)
    def _tail():
        t0 = MAIN + wid * 8
        pltpu.sync_copy(idx_hbm.at[pl.ds(t0 - 8, L)], tidx_v)
        pltpu.sync_copy(x_hbm.at[pl.ds(t0 * D, 8 * D)], txb)
        tv = tidx_v[...]
        for rl in range(8):
            t = _splat_lane(tv, 8 + rl) * D
            _add_row(txb, tbl_v, t, rl * D)
        pltpu.sync_copy(txb, out_hbm.at[pl.ds(t0 * D, 8 * D)])


@jax.jit
def _sc_add_embed(x, idx, tbl):
    mesh = plsc.VectorSubcoreMesh(
        core_axis_name="c", subcore_axis_name="s", num_cores=NC, num_subcores=NS
    )
    return pl.kernel(
        _body,
        out_type=jax.ShapeDtypeStruct((N_NODES * D,), jnp.float32),
        mesh=mesh,
        compiler_params=pltpu.CompilerParams(needs_layout_passes=False),
        scratch_types=[
            pltpu.VMEM((ROWS_W + L,), jnp.int32),
            pltpu.VMEM((64 * D,), jnp.float32),
            pltpu.VMEM((CH * D,), jnp.float32),
            pltpu.VMEM((CH * D,), jnp.float32),
            pltpu.VMEM((CH * D,), jnp.float32),
            pltpu.VMEM((L,), jnp.int32),
            pltpu.VMEM((8 * D,), jnp.float32),
            pltpu.SemaphoreType.DMA,
            pltpu.SemaphoreType.DMA,
            pltpu.SemaphoreType.DMA,
            pltpu.SemaphoreType.DMA,
            pltpu.SemaphoreType.DMA,
            pltpu.SemaphoreType.DMA,
            pltpu.SemaphoreType.DMA,
        ],
    )(x, idx, tbl)


def kernel(x, node_types, type_table):
    out = _sc_add_embed(
        x.reshape(-1), node_types.astype(jnp.int32), type_table.reshape(-1))
    return out.reshape(N_NODES, D)


# speedup vs baseline: 3.5741x; 1.0260x over previous
import jax
import jax.numpy as jnp
from jax import lax
from jax.experimental import pallas as pl
from jax.experimental.pallas import tpu as pltpu
from jax.experimental.pallas import tpu_sc as plsc

N_NODES = 100000
D = 128
NC = 2
NS = 16
NW = NC * NS
L = 16

ROWS_W = 3120
MAIN = NW * ROWS_W
CH = 240
NCHUNK = ROWS_W // CH
NBUF = 3
TAIL = N_NODES - MAIN
TAIL_W = TAIL // 8


def _splat_lane(tv, rl):
    dnums = lax.GatherDimensionNumbers(
        offset_dims=(), collapsed_slice_dims=(0,), start_index_map=(0,))
    return lax.gather(
        tv, jnp.full((L, 1), rl, jnp.int32), dnums, (1,),
        mode=lax.GatherScatterMode.PROMISE_IN_BOUNDS)


def _add_row(xb, tbl_v, t, rowflat):
    for j in range(D // L):
        col = jnp.arange(j * L, (j + 1) * L, dtype=jnp.int32)
        v = plsc.load_gather(tbl_v, [t + col])
        plsc.addupdate(xb.at[pl.ds(rowflat + j * L, L)], v)


def _body(x_hbm, idx_hbm, tbl_hbm, out_hbm, idx_v, tbl_v,
          xb0, xb1, xb2, tidx_v, txb,
          sx0, sx1, sx2, so0, so1, so2, st, stx):
    xb = [xb0, xb1, xb2]
    sx = [sx0, sx1, sx2]
    so = [so0, so1, so2]

    wid = lax.axis_index("s") * NC + lax.axis_index("c")
    base = wid * ROWS_W
    t0 = MAIN + wid * 8

    def start_in(k):
        s = k % NBUF
        return pltpu.async_copy(
            x_hbm.at[pl.ds((base + k * CH) * D, CH * D)], xb[s], sx[s])

    in_desc = [None] * NCHUNK
    out_desc = [None] * NCHUNK
    in_desc[0] = start_in(0)
    in_desc[1] = start_in(1)
    in_desc[2] = start_in(2)
    tail_on = wid < TAIL_W

    @pl.when(tail_on)
    def _tail_prefetch():
        pltpu.async_copy(idx_hbm.at[pl.ds(t0 - 8, L)], tidx_v, st)
        pltpu.async_copy(x_hbm.at[pl.ds(t0 * D, 8 * D)], txb, stx)

    pltpu.sync_copy(tbl_hbm, tbl_v)
    pltpu.sync_copy(idx_hbm.at[pl.ds(base, ROWS_W)], idx_v.at[pl.ds(0, ROWS_W)])

    for k in range(NCHUNK):
        s = k % NBUF
        in_desc[k].wait()

        @plsc.parallel_loop(0, CH, unroll=2)
        def chunk_rows(r, k=k, s=s):
            tv = idx_v[pl.ds(k * CH + r, L)]
            t = _splat_lane(tv, 0) * D
            _add_row(xb[s], tbl_v, t, r * D)

        out_desc[k] = pltpu.async_copy(
            xb[s], out_hbm.at[pl.ds((base + k * CH) * D, CH * D)], so[s])
        if k >= 1 and k + 2 < NCHUNK:
            out_desc[k - 1].wait()
            in_desc[k + 2] = start_in(k + 2)
    @pl.when(tail_on)
    def _tail():
        pltpu.make_async_copy(idx_hbm.at[pl.ds(t0 - 8, L)], tidx_v, st).wait()
        pltpu.make_async_copy(x_hbm.at[pl.ds(t0 * D, 8 * D)], txb, stx).wait()
        tv = tidx_v[...]
        for rl in range(8):
            t = _splat_lane(tv, 8 + rl) * D
            _add_row(txb, tbl_v, t, rl * D)
        pltpu.sync_copy(txb, out_hbm.at[pl.ds(t0 * D, 8 * D)])

    for k in range(max(0, NCHUNK - 3), NCHUNK):
        out_desc[k].wait()


@jax.jit
def _sc_add_embed(x, idx, tbl):
    mesh = plsc.VectorSubcoreMesh(
        core_axis_name="c", subcore_axis_name="s", num_cores=NC, num_subcores=NS
    )
    return pl.kernel(
        _body,
        out_type=jax.ShapeDtypeStruct((N_NODES * D,), jnp.float32),
        mesh=mesh,
        compiler_params=pltpu.CompilerParams(needs_layout_passes=False),
        scratch_types=[
            pltpu.VMEM((ROWS_W + L,), jnp.int32),
            pltpu.VMEM((64 * D,), jnp.float32),
            pltpu.VMEM((CH * D,), jnp.float32),
            pltpu.VMEM((CH * D,), jnp.float32),
            pltpu.VMEM((CH * D,), jnp.float32),
            pltpu.VMEM((L,), jnp.int32),
            pltpu.VMEM((8 * D,), jnp.float32),
            pltpu.SemaphoreType.DMA,
            pltpu.SemaphoreType.DMA,
            pltpu.SemaphoreType.DMA,
            pltpu.SemaphoreType.DMA,
            pltpu.SemaphoreType.DMA,
            pltpu.SemaphoreType.DMA,
            pltpu.SemaphoreType.DMA,
            pltpu.SemaphoreType.DMA,
        ],
    )(x, idx, tbl)


def kernel(x, node_types, type_table):
    out = _sc_add_embed(
        x.reshape(-1), node_types.astype(jnp.int32), type_table.reshape(-1))
    return out.reshape(N_NODES, D)
